# Initial kernel scaffold; baseline (speedup 1.0000x reference)
import jax
import jax.numpy as jnp
from jax.experimental import pallas as pl


def kernel(feat, coords, knn_idx, W, b):
    raise NotImplementedError("write your pallas kernel here")


# speedup vs baseline: 2.6157x; 2.6157x over previous
import functools

import jax
import jax.numpy as jnp
from jax import lax
from jax.experimental import pallas as pl
from jax.experimental.pallas import tpu as pltpu
from jax.experimental.pallas import tpu_sc as plsc

N = 10000
C = 128
K = 32
DAUG = 144
NC = 2
NS = 16
NW = NC * NS
ROWS_W = 320
CH = 4
R = ROWS_W // CH
N_PAD = NW * ROWS_W


def _sc_body(aug_hbm, idx_hbm, sums_hbm, idx_v, acc_v, sem):
    wid = lax.axis_index("s") * NC + lax.axis_index("c")
    pltpu.sync_copy(idx_hbm.at[wid], idx_v)
    for c in range(CH):
        base = wid * ROWS_W + c * R
        pltpu.async_copy(aug_hbm.at[idx_v.at[c]], acc_v, sem).wait()

        def _acc(k, carry):
            row = k * CH + c
            pltpu.async_copy(aug_hbm.at[idx_v.at[row]], acc_v, sem,
                             add=True).wait()
            return carry

        lax.fori_loop(1, K, _acc, 0)
        pltpu.sync_copy(acc_v, sums_hbm.at[pl.ds(base, R)])


def _sc_gather_sums(aug, idx_r):
    mesh = plsc.VectorSubcoreMesh(core_axis_name="c", subcore_axis_name="s")
    return pl.kernel(
        _sc_body,
        out_type=jax.ShapeDtypeStruct((N_PAD, DAUG), jnp.float32),
        mesh=mesh,
        scratch_types=[
            pltpu.VMEM((K * CH, R), jnp.int32),
            pltpu.VMEM((R, DAUG), jnp.float32),
            pltpu.SemaphoreType.DMA,
        ],
        compiler_params=pltpu.CompilerParams(use_tc_tiling_on_sc=False),
    )(aug, idx_r)


def _tc_body(feat_ref, sums_ref, c8_ref, w_ref, b_ref, out_ref):
    f = feat_ref[...]
    s = sums_ref[...]
    c8 = c8_ref[...]
    w = w_ref[...]
    inv = jnp.float32(1.0 / K)
    agg = s[:, :C] * inv
    m1 = s[:, C:C + 3] * inv
    m2 = s[:, C + 3:C + 6] * inv
    rm = m1 - c8[:, :3]
    rs = jnp.sqrt(jnp.maximum(m2 - m1 * m1, 0.0))
    rel = jnp.concatenate([rm, rs], axis=1)
    y = (jnp.dot(f, w[:C], preferred_element_type=jnp.float32)
         + jnp.dot(agg, w[C:2 * C], preferred_element_type=jnp.float32)
         + jnp.dot(rel, w[2 * C:2 * C + 6], preferred_element_type=jnp.float32)
         + b_ref[...])
    out_ref[...] = y * jax.nn.sigmoid(y)


def _tc_dense(feat_p, sums, c8_p, w, b, interpret=False):
    br = 1024
    grid = (N_PAD // br,)
    return pl.pallas_call(
        _tc_body,
        grid=grid,
        in_specs=[
            pl.BlockSpec((br, C), lambda i: (i, 0)),
            pl.BlockSpec((br, DAUG), lambda i: (i, 0)),
            pl.BlockSpec((br, 8), lambda i: (i, 0)),
            pl.BlockSpec((2 * C + 6, C), lambda i: (0, 0)),
            pl.BlockSpec((1, C), lambda i: (0, 0)),
        ],
        out_specs=pl.BlockSpec((br, C), lambda i: (i, 0)),
        out_shape=jax.ShapeDtypeStruct((N_PAD, C), jnp.float32),
        interpret=interpret,
    )(feat_p, sums, c8_p, w, b)


def kernel(feat, coords, knn_idx, W, b):
    feat = feat.astype(jnp.float32)
    coords = coords.astype(jnp.float32)
    idx32 = knn_idx.astype(jnp.int32)

    aug = jnp.concatenate(
        [feat, coords, coords * coords,
         jnp.zeros((N, DAUG - C - 6), jnp.float32)], axis=1)

    idx_pad = jnp.pad(idx32, ((0, N_PAD - N), (0, 0)))
    idx_r = (idx_pad.reshape(NW, CH, R, K)
             .transpose(0, 3, 1, 2)
             .reshape(NW, K * CH, R))

    sums = _sc_gather_sums(aug, idx_r)

    feat_p = jnp.pad(feat, ((0, N_PAD - N), (0, 0)))
    c8 = jnp.pad(coords, ((0, N_PAD - N), (0, 5)))
    out = _tc_dense(feat_p, sums, c8, W.astype(jnp.float32),
                    b.astype(jnp.float32).reshape(1, C))
    return out[:N]


# speedup vs baseline: 2.7715x; 1.0596x over previous
import functools

import jax
import jax.numpy as jnp
from jax import lax
from jax.experimental import pallas as pl
from jax.experimental.pallas import tpu as pltpu
from jax.experimental.pallas import tpu_sc as plsc

N = 10000
C = 128
K = 32
DAUG = 144
NC = 2
NS = 16
NW = NC * NS
ROWS_W = 320
CH = 4
R = ROWS_W // CH
N_PAD = NW * ROWS_W


def _sc_body(aug_hbm, idx_hbm, sums_hbm, idx_v, acc_v, sem):
    wid = lax.axis_index("s") * NC + lax.axis_index("c")
    pltpu.sync_copy(idx_hbm.at[wid], idx_v)
    for c in range(CH):
        base = wid * ROWS_W + c * R
        pltpu.async_copy(aug_hbm.at[idx_v.at[c]], acc_v, sem).wait()

        def _fire(k, carry):
            pltpu.async_copy(aug_hbm.at[idx_v.at[k * CH + c]], acc_v, sem,
                             add=True)
            return carry

        def _drain(k, carry):
            pltpu.make_async_copy(aug_hbm.at[idx_v.at[k * CH + c]], acc_v,
                                  sem).wait()
            return carry

        lax.fori_loop(1, K, _fire, 0)
        lax.fori_loop(1, K, _drain, 0)
        pltpu.sync_copy(acc_v, sums_hbm.at[pl.ds(base, R)])


def _sc_gather_sums(aug, idx_r):
    mesh = plsc.VectorSubcoreMesh(core_axis_name="c", subcore_axis_name="s")
    return pl.kernel(
        _sc_body,
        out_type=jax.ShapeDtypeStruct((N_PAD, DAUG), jnp.float32),
        mesh=mesh,
        scratch_types=[
            pltpu.VMEM((K * CH, R), jnp.int32),
            pltpu.VMEM((R, DAUG), jnp.float32),
            pltpu.SemaphoreType.DMA,
        ],
        compiler_params=pltpu.CompilerParams(use_tc_tiling_on_sc=False),
    )(aug, idx_r)


def _tc_body(feat_ref, sums_ref, c8_ref, w_ref, b_ref, out_ref):
    f = feat_ref[...]
    s = sums_ref[...]
    c8 = c8_ref[...]
    w = w_ref[...]
    inv = jnp.float32(1.0 / K)
    agg = s[:, :C] * inv
    m1 = s[:, C:C + 3] * inv
    m2 = s[:, C + 3:C + 6] * inv
    rm = m1 - c8[:, :3]
    rs = jnp.sqrt(jnp.maximum(m2 - m1 * m1, 0.0))
    rel = jnp.concatenate([rm, rs], axis=1)
    y = (jnp.dot(f, w[:C], preferred_element_type=jnp.float32)
         + jnp.dot(agg, w[C:2 * C], preferred_element_type=jnp.float32)
         + jnp.dot(rel, w[2 * C:2 * C + 6], preferred_element_type=jnp.float32)
         + b_ref[...])
    out_ref[...] = y * jax.nn.sigmoid(y)


def _tc_dense(feat_p, sums, c8_p, w, b, interpret=False):
    br = 1024
    grid = (N_PAD // br,)
    return pl.pallas_call(
        _tc_body,
        grid=grid,
        in_specs=[
            pl.BlockSpec((br, C), lambda i: (i, 0)),
            pl.BlockSpec((br, DAUG), lambda i: (i, 0)),
            pl.BlockSpec((br, 8), lambda i: (i, 0)),
            pl.BlockSpec((2 * C + 6, C), lambda i: (0, 0)),
            pl.BlockSpec((1, C), lambda i: (0, 0)),
        ],
        out_specs=pl.BlockSpec((br, C), lambda i: (i, 0)),
        out_shape=jax.ShapeDtypeStruct((N_PAD, C), jnp.float32),
        interpret=interpret,
    )(feat_p, sums, c8_p, w, b)


def kernel(feat, coords, knn_idx, W, b):
    feat = feat.astype(jnp.float32)
    coords = coords.astype(jnp.float32)
    idx32 = knn_idx.astype(jnp.int32)

    aug = jnp.concatenate(
        [feat, coords, coords * coords,
         jnp.zeros((N, DAUG - C - 6), jnp.float32)], axis=1)

    idx_pad = jnp.pad(idx32, ((0, N_PAD - N), (0, 0)))
    idx_r = (idx_pad.reshape(NW, CH, R, K)
             .transpose(0, 3, 1, 2)
             .reshape(NW, K * CH, R))

    sums = _sc_gather_sums(aug, idx_r)

    feat_p = jnp.pad(feat, ((0, N_PAD - N), (0, 0)))
    c8 = jnp.pad(coords, ((0, N_PAD - N), (0, 5)))
    out = _tc_dense(feat_p, sums, c8, W.astype(jnp.float32),
                    b.astype(jnp.float32).reshape(1, C))
    return out[:N]


# speedup vs baseline: 8.6618x; 3.1253x over previous
import functools

import jax
import jax.numpy as jnp
from jax import lax
from jax.experimental import pallas as pl
from jax.experimental.pallas import tpu as pltpu
from jax.experimental.pallas import tpu_sc as plsc

N = 10000
C = 128
K = 32
DAUG = 144
NC = 2
NS = 16
NW = NC * NS
ROWS_W = 320
CH = 4
R = ROWS_W // CH
N_PAD = NW * ROWS_W


def _sc_body(aug_hbm, idx_hbm, sums_hbm, idx_v, acc_v, aug_sh, sem):
    sid = lax.axis_index("s")
    wid = sid * NC + lax.axis_index("c")
    @pl.when(sid == 0)
    def _load_table():
        pltpu.sync_copy(aug_hbm, aug_sh)

    pltpu.sync_copy(idx_hbm.at[wid], idx_v)
    plsc.subcore_barrier()
    for c in range(CH):
        base = wid * ROWS_W + c * R
        pltpu.async_copy(aug_sh.at[idx_v.at[c]], acc_v, sem).wait()

        def _fire(k, carry):
            pltpu.async_copy(aug_sh.at[idx_v.at[k * CH + c]], acc_v, sem,
                             add=True)
            return carry

        def _drain(k, carry):
            pltpu.make_async_copy(aug_sh.at[idx_v.at[k * CH + c]], acc_v,
                                  sem).wait()
            return carry

        lax.fori_loop(1, K, _fire, 0)
        lax.fori_loop(1, K, _drain, 0)
        pltpu.sync_copy(acc_v, sums_hbm.at[pl.ds(base, R)])


def _sc_gather_sums(aug, idx_r):
    mesh = plsc.VectorSubcoreMesh(core_axis_name="c", subcore_axis_name="s")
    return pl.kernel(
        _sc_body,
        out_type=jax.ShapeDtypeStruct((N_PAD, DAUG), jnp.float32),
        mesh=mesh,
        scratch_types=[
            pltpu.VMEM((K * CH, R), jnp.int32),
            pltpu.VMEM((R, DAUG), jnp.float32),
            pltpu.VMEM_SHARED((N, DAUG), jnp.float32),
            pltpu.SemaphoreType.DMA,
        ],
        compiler_params=pltpu.CompilerParams(use_tc_tiling_on_sc=False),
    )(aug, idx_r)


def _tc_body(feat_ref, sums_ref, c8_ref, w_ref, b_ref, out_ref):
    f = feat_ref[...]
    s = sums_ref[...]
    c8 = c8_ref[...]
    w = w_ref[...]
    inv = jnp.float32(1.0 / K)
    agg = s[:, :C] * inv
    m1 = s[:, C:C + 3] * inv
    m2 = s[:, C + 3:C + 6] * inv
    rm = m1 - c8[:, :3]
    rs = jnp.sqrt(jnp.maximum(m2 - m1 * m1, 0.0))
    rel = jnp.concatenate([rm, rs], axis=1)
    y = (jnp.dot(f, w[:C], preferred_element_type=jnp.float32)
         + jnp.dot(agg, w[C:2 * C], preferred_element_type=jnp.float32)
         + jnp.dot(rel, w[2 * C:2 * C + 6], preferred_element_type=jnp.float32)
         + b_ref[...])
    out_ref[...] = y * jax.nn.sigmoid(y)


def _tc_dense(feat_p, sums, c8_p, w, b, interpret=False):
    br = 1024
    grid = (N_PAD // br,)
    return pl.pallas_call(
        _tc_body,
        grid=grid,
        in_specs=[
            pl.BlockSpec((br, C), lambda i: (i, 0)),
            pl.BlockSpec((br, DAUG), lambda i: (i, 0)),
            pl.BlockSpec((br, 8), lambda i: (i, 0)),
            pl.BlockSpec((2 * C + 6, C), lambda i: (0, 0)),
            pl.BlockSpec((1, C), lambda i: (0, 0)),
        ],
        out_specs=pl.BlockSpec((br, C), lambda i: (i, 0)),
        out_shape=jax.ShapeDtypeStruct((N_PAD, C), jnp.float32),
        interpret=interpret,
    )(feat_p, sums, c8_p, w, b)


def kernel(feat, coords, knn_idx, W, b):
    feat = feat.astype(jnp.float32)
    coords = coords.astype(jnp.float32)
    idx32 = knn_idx.astype(jnp.int32)

    aug = jnp.concatenate(
        [feat, coords, coords * coords,
         jnp.zeros((N, DAUG - C - 6), jnp.float32)], axis=1)

    idx_pad = jnp.pad(idx32, ((0, N_PAD - N), (0, 0)))
    idx_r = (idx_pad.reshape(NW, CH, R, K)
             .transpose(0, 3, 1, 2)
             .reshape(NW, K * CH, R))

    sums = _sc_gather_sums(aug, idx_r)

    feat_p = jnp.pad(feat, ((0, N_PAD - N), (0, 0)))
    c8 = jnp.pad(coords, ((0, N_PAD - N), (0, 5)))
    out = _tc_dense(feat_p, sums, c8, W.astype(jnp.float32),
                    b.astype(jnp.float32).reshape(1, C))
    return out[:N]


# speedup vs baseline: 11.2823x; 1.3025x over previous
import jax
import jax.numpy as jnp
from jax import lax
from jax.experimental import pallas as pl
from jax.experimental.pallas import tpu as pltpu
from jax.experimental.pallas import tpu_sc as plsc

N = 10000
C = 128
K = 32
DC = 8
NC = 2
NS = 16
NW = NC * NS
ROWS_W = 320
CH = 4
R = ROWS_W // CH
N_PAD = NW * ROWS_W


def _sc_body(feat_hbm, c8_hbm, idx_hbm, sumsf_hbm, sumsc_hbm,
             idx_v, accf_v, accc_v, feat_sh, c8_sh, sem):
    sid = lax.axis_index("s")
    wid = sid * NC + lax.axis_index("c")
    @pl.when(sid == 0)
    def _load_tables():
        pltpu.sync_copy(feat_hbm, feat_sh)
        pltpu.sync_copy(c8_hbm, c8_sh)

    pltpu.sync_copy(idx_hbm.at[wid], idx_v)
    plsc.subcore_barrier()
    for c in range(CH):
        base = wid * ROWS_W + c * R
        f0 = pltpu.async_copy(feat_sh.at[idx_v.at[c]], accf_v, sem)
        pltpu.async_copy(c8_sh.at[idx_v.at[c]], accc_v, sem)
        f0.wait()
        pltpu.make_async_copy(c8_sh.at[idx_v.at[c]], accc_v, sem).wait()

        def _fire(k, carry):
            row = k * CH + c
            pltpu.async_copy(feat_sh.at[idx_v.at[row]], accf_v, sem,
                             add=True)
            pltpu.async_copy(c8_sh.at[idx_v.at[row]], accc_v, sem,
                             add=True)
            return carry

        def _drain(k, carry):
            row = k * CH + c
            pltpu.make_async_copy(feat_sh.at[idx_v.at[row]], accf_v,
                                  sem).wait()
            pltpu.make_async_copy(c8_sh.at[idx_v.at[row]], accc_v,
                                  sem).wait()
            return carry

        lax.fori_loop(1, K, _fire, 0)
        lax.fori_loop(1, K, _drain, 0)
        pltpu.sync_copy(accf_v, sumsf_hbm.at[pl.ds(base, R)])
        pltpu.sync_copy(accc_v, sumsc_hbm.at[pl.ds(base, R)])


def _sc_gather_sums(feat, c8, idx_r):
    mesh = plsc.VectorSubcoreMesh(core_axis_name="c", subcore_axis_name="s")
    return pl.kernel(
        _sc_body,
        out_type=(jax.ShapeDtypeStruct((N_PAD, C), jnp.float32),
                  jax.ShapeDtypeStruct((N_PAD, DC), jnp.float32)),
        mesh=mesh,
        scratch_types=[
            pltpu.VMEM((K * CH, R), jnp.int32),
            pltpu.VMEM((R, C), jnp.float32),
            pltpu.VMEM((R, DC), jnp.float32),
            pltpu.VMEM_SHARED((N, C), jnp.float32),
            pltpu.VMEM_SHARED((N, DC), jnp.float32),
            pltpu.SemaphoreType.DMA,
        ],
        compiler_params=pltpu.CompilerParams(use_tc_tiling_on_sc=False),
    )(feat, c8, idx_r)


def _tc_body(feat_ref, sumsf_ref, sumsc_ref, c8_ref, w_ref, b_ref, out_ref):
    f = feat_ref[...]
    sc_ = sumsc_ref[...]
    c8 = c8_ref[...]
    w = w_ref[...]
    inv = jnp.float32(1.0 / K)
    agg = sumsf_ref[...] * inv
    m1 = sc_[:, 0:3] * inv
    m2 = sc_[:, 3:6] * inv
    rm = m1 - c8[:, 0:3]
    rs = jnp.sqrt(jnp.maximum(m2 - m1 * m1, 0.0))
    rel = jnp.concatenate([rm, rs], axis=1)
    y = (jnp.dot(f, w[:C], preferred_element_type=jnp.float32)
         + jnp.dot(agg, w[C:2 * C], preferred_element_type=jnp.float32)
         + jnp.dot(rel, w[2 * C:2 * C + 6], preferred_element_type=jnp.float32)
         + b_ref[...])
    out_ref[...] = y * jax.nn.sigmoid(y)


def _tc_dense(feat, sumsf, sumsc, c8, w, b, interpret=False):
    br = 1000
    grid = (N // br,)
    return pl.pallas_call(
        _tc_body,
        grid=grid,
        in_specs=[
            pl.BlockSpec((br, C), lambda i: (i, 0)),
            pl.BlockSpec((br, C), lambda i: (i, 0)),
            pl.BlockSpec((br, DC), lambda i: (i, 0)),
            pl.BlockSpec((br, DC), lambda i: (i, 0)),
            pl.BlockSpec((2 * C + 6, C), lambda i: (0, 0)),
            pl.BlockSpec((1, C), lambda i: (0, 0)),
        ],
        out_specs=pl.BlockSpec((br, C), lambda i: (i, 0)),
        out_shape=jax.ShapeDtypeStruct((N, C), jnp.float32),
        interpret=interpret,
    )(feat, sumsf, sumsc, c8, w, b)


def kernel(feat, coords, knn_idx, W, b):
    feat = feat.astype(jnp.float32)
    coords = coords.astype(jnp.float32)
    idx32 = knn_idx.astype(jnp.int32)

    c8 = jnp.concatenate(
        [coords, coords * coords, jnp.zeros((N, DC - 6), jnp.float32)],
        axis=1)

    idx_pad = jnp.pad(idx32, ((0, N_PAD - N), (0, 0)))
    idx_r = (idx_pad.reshape(NW, CH, R, K)
             .transpose(0, 3, 1, 2)
             .reshape(NW, K * CH, R))

    sumsf, sumsc = _sc_gather_sums(feat, c8, idx_r)

    return _tc_dense(feat, sumsf, sumsc, c8, W.astype(jnp.float32),
                     b.astype(jnp.float32).reshape(1, C))


# speedup vs baseline: 11.3904x; 1.0096x over previous
import jax
import jax.numpy as jnp
from jax import lax
from jax.experimental import pallas as pl
from jax.experimental.pallas import tpu as pltpu
from jax.experimental.pallas import tpu_sc as plsc

N = 10000
C = 128
K = 32
DC = 8
NC = 2
NS = 16
NW = NC * NS
ROWS_W = 320
CH = 4
R = ROWS_W // CH
N_PAD = NW * ROWS_W
NBUF = 3


def _sc_body(feat_hbm, c8_hbm, idx_hbm, sumsf_hbm, sumsc_hbm,
             idx_v, accf_v, accc_v, feat_sh, c8_sh, semg, semi):
    sid = lax.axis_index("s")
    wid = sid * NC + lax.axis_index("c")
    rows16 = N // NS
    sl = pl.ds(sid * rows16, rows16)
    pltpu.sync_copy(feat_hbm.at[sl], feat_sh.at[sl])
    pltpu.sync_copy(c8_hbm.at[sl], c8_sh.at[sl])
    pltpu.sync_copy(idx_hbm.at[wid], idx_v)
    plsc.subcore_barrier()

    def _drain_wb(c):
        buf = c % NBUF

        def _drain(k, carry):
            row = k * CH + c
            pltpu.make_async_copy(feat_sh.at[idx_v.at[row]],
                                  accf_v.at[buf], semg).wait()
            pltpu.make_async_copy(c8_sh.at[idx_v.at[row]],
                                  accc_v.at[buf], semg).wait()
            return carry

        lax.fori_loop(1, K, _drain, 0)
        base = wid * ROWS_W + c * R
        pltpu.sync_copy(accf_v.at[buf], sumsf_hbm.at[pl.ds(base, R)])
        pltpu.sync_copy(accc_v.at[buf], sumsc_hbm.at[pl.ds(base, R)])

    for c in range(CH):
        buf = c % NBUF
        if c >= NBUF:
            _drain_wb(c - NBUF)
        f0 = pltpu.async_copy(feat_sh.at[idx_v.at[c]], accf_v.at[buf], semi)
        pltpu.async_copy(c8_sh.at[idx_v.at[c]], accc_v.at[buf], semi)
        f0.wait()
        pltpu.make_async_copy(c8_sh.at[idx_v.at[c]], accc_v.at[buf],
                              semi).wait()

        def _fire(k, carry):
            row = k * CH + c
            pltpu.async_copy(feat_sh.at[idx_v.at[row]], accf_v.at[buf],
                             semg, add=True)
            pltpu.async_copy(c8_sh.at[idx_v.at[row]], accc_v.at[buf],
                             semg, add=True)
            return carry

        lax.fori_loop(1, K, _fire, 0)

    for c in range(CH - NBUF, CH):
        _drain_wb(c)


def _sc_gather_sums(feat, c8, idx_r):
    mesh = plsc.VectorSubcoreMesh(core_axis_name="c", subcore_axis_name="s")
    return pl.kernel(
        _sc_body,
        out_type=(jax.ShapeDtypeStruct((N_PAD, C), jnp.float32),
                  jax.ShapeDtypeStruct((N_PAD, DC), jnp.float32)),
        mesh=mesh,
        scratch_types=[
            pltpu.VMEM((K * CH, R), jnp.int32),
            pltpu.VMEM((NBUF, R, C), jnp.float32),
            pltpu.VMEM((NBUF, R, DC), jnp.float32),
            pltpu.VMEM_SHARED((N, C), jnp.float32),
            pltpu.VMEM_SHARED((N, DC), jnp.float32),
            pltpu.SemaphoreType.DMA,
            pltpu.SemaphoreType.DMA,
        ],
        compiler_params=pltpu.CompilerParams(use_tc_tiling_on_sc=False),
    )(feat, c8, idx_r)


def _tc_body(feat_ref, sumsf_ref, sumsc_ref, c8_ref, w_ref, b_ref, out_ref):
    f = feat_ref[...]
    sc_ = sumsc_ref[...]
    c8 = c8_ref[...]
    w = w_ref[...]
    inv = jnp.float32(1.0 / K)
    agg = sumsf_ref[...] * inv
    m1 = sc_[:, 0:3] * inv
    m2 = sc_[:, 3:6] * inv
    rm = m1 - c8[:, 0:3]
    rs = jnp.sqrt(jnp.maximum(m2 - m1 * m1, 0.0))
    rel = jnp.concatenate([rm, rs], axis=1)
    y = (jnp.dot(f, w[:C], preferred_element_type=jnp.float32)
         + jnp.dot(agg, w[C:2 * C], preferred_element_type=jnp.float32)
         + jnp.dot(rel, w[2 * C:2 * C + 6], preferred_element_type=jnp.float32)
         + b_ref[...])
    out_ref[...] = y * jax.nn.sigmoid(y)


def _tc_dense(feat, sumsf, sumsc, c8, w, b, interpret=False):
    br = 1000
    grid = (N // br,)
    return pl.pallas_call(
        _tc_body,
        grid=grid,
        in_specs=[
            pl.BlockSpec((br, C), lambda i: (i, 0)),
            pl.BlockSpec((br, C), lambda i: (i, 0)),
            pl.BlockSpec((br, DC), lambda i: (i, 0)),
            pl.BlockSpec((br, DC), lambda i: (i, 0)),
            pl.BlockSpec((2 * C + 6, C), lambda i: (0, 0)),
            pl.BlockSpec((1, C), lambda i: (0, 0)),
        ],
        out_specs=pl.BlockSpec((br, C), lambda i: (i, 0)),
        out_shape=jax.ShapeDtypeStruct((N, C), jnp.float32),
        interpret=interpret,
    )(feat, sumsf, sumsc, c8, w, b)


def kernel(feat, coords, knn_idx, W, b):
    feat = feat.astype(jnp.float32)
    coords = coords.astype(jnp.float32)
    idx32 = knn_idx.astype(jnp.int32)

    c8 = jnp.concatenate(
        [coords, coords * coords, jnp.zeros((N, DC - 6), jnp.float32)],
        axis=1)

    idx_pad = jnp.pad(idx32, ((0, N_PAD - N), (0, 0)))
    idx_r = (idx_pad.reshape(NW, CH, R, K)
             .transpose(0, 3, 1, 2)
             .reshape(NW, K * CH, R))

    sumsf, sumsc = _sc_gather_sums(feat, c8, idx_r)

    return _tc_dense(feat, sumsf, sumsc, c8, W.astype(jnp.float32),
                     b.astype(jnp.float32).reshape(1, C))


# speedup vs baseline: 11.7622x; 1.0326x over previous
import jax
import jax.numpy as jnp
from jax import lax
from jax.experimental import pallas as pl
from jax.experimental.pallas import tpu as pltpu
from jax.experimental.pallas import tpu_sc as plsc

N = 10000
C = 128
K = 32
DC = 8
NC = 2
NS = 16
NW = NC * NS
ROWS_W = 320
CH = 4
R = ROWS_W // CH
N_PAD = NW * ROWS_W
NBUF = 3


def _sc_body(feat_hbm, c8_hbm, idx_hbm, sumsf_hbm, sumsc_hbm,
             idx_v, accf_v, accc_v, feat_sh, c8_sh, semg, semi):
    sid = lax.axis_index("s")
    wid = sid * NC + lax.axis_index("c")
    rows16 = N // NS
    sl = pl.ds(sid * rows16, rows16)
    pltpu.sync_copy(feat_hbm.at[sl], feat_sh.at[sl])
    pltpu.sync_copy(c8_hbm.at[sl], c8_sh.at[sl])
    pltpu.sync_copy(idx_hbm.at[wid], idx_v)
    plsc.subcore_barrier()

    def _drain_wb(c):
        buf = c % NBUF

        def _drain(k, carry):
            row = k * CH + c
            pltpu.make_async_copy(feat_sh.at[idx_v.at[row]],
                                  accf_v.at[buf], semg).wait()
            pltpu.make_async_copy(c8_sh.at[idx_v.at[row]],
                                  accc_v.at[buf], semg).wait()
            return carry

        lax.fori_loop(1, K, _drain, 0)
        base = wid * ROWS_W + c * R
        pltpu.sync_copy(accf_v.at[buf], sumsf_hbm.at[pl.ds(base, R)])
        pltpu.sync_copy(accc_v.at[buf], sumsc_hbm.at[pl.ds(base, R)])

    for c in range(CH):
        buf = c % NBUF
        if c >= NBUF:
            _drain_wb(c - NBUF)
        f0 = pltpu.async_copy(feat_sh.at[idx_v.at[c]], accf_v.at[buf], semi)
        pltpu.async_copy(c8_sh.at[idx_v.at[c]], accc_v.at[buf], semi)
        f0.wait()
        pltpu.make_async_copy(c8_sh.at[idx_v.at[c]], accc_v.at[buf],
                              semi).wait()

        def _fire(k, carry):
            row = k * CH + c
            pltpu.async_copy(feat_sh.at[idx_v.at[row]], accf_v.at[buf],
                             semg, add=True)
            pltpu.async_copy(c8_sh.at[idx_v.at[row]], accc_v.at[buf],
                             semg, add=True)
            return carry

        lax.fori_loop(1, K, _fire, 0)

    for c in range(CH - NBUF, CH):
        _drain_wb(c)


def _sc_gather_sums(feat, c8, idx_r):
    mesh = plsc.VectorSubcoreMesh(core_axis_name="c", subcore_axis_name="s")
    return pl.kernel(
        _sc_body,
        out_type=(jax.ShapeDtypeStruct((N_PAD, C), jnp.bfloat16),
                  jax.ShapeDtypeStruct((N_PAD, DC), jnp.float32)),
        mesh=mesh,
        scratch_types=[
            pltpu.VMEM((K * CH, R), jnp.int32),
            pltpu.VMEM((NBUF, R, C), jnp.bfloat16),
            pltpu.VMEM((NBUF, R, DC), jnp.float32),
            pltpu.VMEM_SHARED((N, C), jnp.bfloat16),
            pltpu.VMEM_SHARED((N, DC), jnp.float32),
            pltpu.SemaphoreType.DMA,
            pltpu.SemaphoreType.DMA,
        ],
        compiler_params=pltpu.CompilerParams(use_tc_tiling_on_sc=False),
    )(feat, c8, idx_r)


def _tc_body(feat_ref, sumsf_ref, sumsc_ref, c8_ref, w_ref, b_ref, out_ref):
    f = feat_ref[...]
    sc_ = sumsc_ref[...]
    c8 = c8_ref[...]
    w = w_ref[...]
    inv = jnp.float32(1.0 / K)
    agg = sumsf_ref[...].astype(jnp.float32) * inv
    m1 = sc_[:, 0:3] * inv
    m2 = sc_[:, 3:6] * inv
    rm = m1 - c8[:, 0:3]
    rs = jnp.sqrt(jnp.maximum(m2 - m1 * m1, 0.0))
    rel = jnp.concatenate([rm, rs], axis=1)
    y = (jnp.dot(f, w[:C], preferred_element_type=jnp.float32)
         + jnp.dot(agg, w[C:2 * C], preferred_element_type=jnp.float32)
         + jnp.dot(rel, w[2 * C:2 * C + 6], preferred_element_type=jnp.float32)
         + b_ref[...])
    out_ref[...] = y * jax.nn.sigmoid(y)


def _tc_dense(feat, sumsf, sumsc, c8, w, b, interpret=False):
    br = 1000
    grid = (N // br,)
    return pl.pallas_call(
        _tc_body,
        grid=grid,
        in_specs=[
            pl.BlockSpec((br, C), lambda i: (i, 0)),
            pl.BlockSpec((br, C), lambda i: (i, 0)),
            pl.BlockSpec((br, DC), lambda i: (i, 0)),
            pl.BlockSpec((br, DC), lambda i: (i, 0)),
            pl.BlockSpec((2 * C + 6, C), lambda i: (0, 0)),
            pl.BlockSpec((1, C), lambda i: (0, 0)),
        ],
        out_specs=pl.BlockSpec((br, C), lambda i: (i, 0)),
        out_shape=jax.ShapeDtypeStruct((N, C), jnp.float32),
        interpret=interpret,
    )(feat, sumsf, sumsc, c8, w, b)


def kernel(feat, coords, knn_idx, W, b):
    feat = feat.astype(jnp.float32)
    coords = coords.astype(jnp.float32)
    idx32 = knn_idx.astype(jnp.int32)

    c8 = jnp.concatenate(
        [coords, coords * coords, jnp.zeros((N, DC - 6), jnp.float32)],
        axis=1)

    idx_pad = jnp.pad(idx32, ((0, N_PAD - N), (0, 0)))
    idx_r = (idx_pad.reshape(NW, CH, R, K)
             .transpose(0, 3, 1, 2)
             .reshape(NW, K * CH, R))

    sumsf, sumsc = _sc_gather_sums(feat.astype(jnp.bfloat16), c8, idx_r)

    return _tc_dense(feat, sumsf, sumsc, c8, W.astype(jnp.float32),
                     b.astype(jnp.float32).reshape(1, C))
